# baseline (speedup 1.0000x reference)
import functools

import jax
import jax.numpy as jnp
from jax.experimental import pallas as pl
from jax.experimental.pallas import tpu as pltpu
from jax.experimental.pallas import tpu_sc as plsc


def _sc_gather(table, idx, b_pad, emb_dim):
    info = plsc.get_sparse_core_info()
    nc, ns = info.num_cores, info.num_subcores
    nw = nc * ns
    b_per_w = b_pad // nw
    n_rows = idx.shape[0]
    mesh = plsc.VectorSubcoreMesh(core_axis_name="c", subcore_axis_name="s")

    @functools.partial(
        pl.kernel,
        mesh=mesh,
        out_type=jax.ShapeDtypeStruct((n_rows, emb_dim), jnp.float32),
        scratch_types=[
            pltpu.VMEM((b_per_w,), jnp.int32),
            pltpu.VMEM((b_per_w, emb_dim), jnp.float32),
            pltpu.SemaphoreType.DMA,
        ],
    )
    def gather_k(table_hbm, idx_hbm, out_hbm, idx_v, rows_v, sem):
        wid = jax.lax.axis_index("s") * nc + jax.lax.axis_index("c")
        base = wid * b_per_w

        @pl.when(base < n_rows)
        def _():
            pltpu.sync_copy(idx_hbm.at[pl.ds(base, b_per_w)], idx_v)
            pltpu.async_copy(table_hbm.at[idx_v], rows_v, sem).wait()
            pltpu.sync_copy(rows_v, out_hbm.at[pl.ds(base, b_per_w)])

    return gather_k(table, idx)


def _mlp_body(nb1, nb2, x_ref, w1_ref, b1_ref, w2_ref, b2_ref, out_ref,
              h_ref, acc_ref):
    i = pl.program_id(0)

    @pl.when(i < nb1)
    def _():
        part = jax.lax.dot_general(
            x_ref[...], w1_ref[...], (((1,), (1,)), ((), ())),
            preferred_element_type=jnp.float32)

        @pl.when(i == 0)
        def _():
            h_ref[...] = part + b1_ref[...]

        @pl.when(i > 0)
        def _():
            h_ref[...] += part

        @pl.when(i == nb1 - 1)
        def _():
            h_ref[...] = jnp.maximum(h_ref[...], 0.0)

    @pl.when(i >= nb1)
    def _():
        j = i - nb1
        logits = jax.lax.dot_general(
            h_ref[...], w2_ref[...], (((1,), (1,)), ((), ())),
            preferred_element_type=jnp.float32) + b2_ref[0]
        out_ref[pl.ds(j, 1)] = logits[None]
        bm = jnp.max(logits)
        bs = jnp.sum(jnp.exp(logits - bm))

        @pl.when(j == 0)
        def _():
            acc_ref[0] = bm
            acc_ref[1] = bs

        @pl.when(j > 0)
        def _():
            m_old = acc_ref[0]
            s_old = acc_ref[1]
            m_new = jnp.maximum(m_old, bm)
            acc_ref[0] = m_new
            acc_ref[1] = s_old * jnp.exp(m_old - m_new) + bs * jnp.exp(bm - m_new)

        @pl.when(j == nb2 - 1)
        def _():
            lse = jnp.log(acc_ref[1]) + acc_ref[0]
            out_ref[...] = out_ref[...] - lse


def _mlp(emb_flat, W1, b1, W2, b2, nb1=8, nb2=20):
    hid, k = W1.shape
    vocab = W2.shape[0]
    bk = k // nb1
    bv = vocab // nb2

    out = pl.pallas_call(
        functools.partial(_mlp_body, nb1, nb2),
        grid=(nb1 + nb2,),
        in_specs=[
            pl.BlockSpec((1, bk), lambda i: (0, jnp.minimum(i, nb1 - 1))),
            pl.BlockSpec((hid, bk), lambda i: (0, jnp.minimum(i, nb1 - 1))),
            pl.BlockSpec((1, hid), lambda i: (0, 0)),
            pl.BlockSpec((bv, hid), lambda i: (jnp.maximum(i - nb1, 0), 0)),
            pl.BlockSpec((1, 1, bv), lambda i: (jnp.maximum(i - nb1, 0), 0, 0)),
        ],
        out_specs=pl.BlockSpec((nb2, 1, bv), lambda i: (0, 0, 0)),
        out_shape=jax.ShapeDtypeStruct((nb2, 1, bv), jnp.float32),
        scratch_shapes=[
            pltpu.VMEM((1, hid), jnp.float32),
            pltpu.SMEM((2,), jnp.float32),
        ],
    )(emb_flat, W1, b1.reshape(1, hid), W2, b2.reshape(nb2, 1, bv))
    return out.reshape(1, vocab)


def kernel(words, table, W1, b1, W2, b2):
    ctx = words.shape[0]
    emb_dim = table.shape[1]
    emb = _sc_gather(table, words.astype(jnp.int32), 256, emb_dim)
    emb_flat = emb.reshape(1, ctx * emb_dim)
    return _mlp(emb_flat, W1, b1, W2, b2, nb1=8, nb2=5)


# speedup vs baseline: 1.0422x; 1.0422x over previous
import functools

import jax
import jax.numpy as jnp
from jax.experimental import pallas as pl
from jax.experimental.pallas import tpu as pltpu
from jax.experimental.pallas import tpu_sc as plsc


def _sc_gather(table, idx, b_pad, emb_dim):
    info = plsc.get_sparse_core_info()
    nc, ns = info.num_cores, info.num_subcores
    nw = nc * ns
    b_per_w = b_pad // nw
    n_rows = idx.shape[0]
    mesh = plsc.VectorSubcoreMesh(core_axis_name="c", subcore_axis_name="s")

    @functools.partial(
        pl.kernel,
        mesh=mesh,
        out_type=jax.ShapeDtypeStruct((n_rows, emb_dim), jnp.float32),
        scratch_types=[
            pltpu.VMEM((b_per_w,), jnp.int32),
            pltpu.VMEM((b_per_w, emb_dim), jnp.float32),
            pltpu.SemaphoreType.DMA,
        ],
    )
    def gather_k(table_hbm, idx_hbm, out_hbm, idx_v, rows_v, sem):
        wid = jax.lax.axis_index("s") * nc + jax.lax.axis_index("c")
        base = wid * b_per_w

        @pl.when(base < n_rows)
        def _():
            pltpu.sync_copy(idx_hbm.at[pl.ds(base, b_per_w)], idx_v)
            pltpu.async_copy(table_hbm.at[idx_v], rows_v, sem).wait()
            pltpu.sync_copy(rows_v, out_hbm.at[pl.ds(base, b_per_w)])

    return gather_k(table, idx)


def _mlp_body(nb1, nb2, x_ref, w1_ref, b1_ref, w2_ref, b2_ref, out_ref,
              h_ref, acc_ref):
    i = pl.program_id(0)

    @pl.when(i < nb1)
    def _():
        part = jax.lax.dot_general(
            x_ref[...], w1_ref[...], (((1,), (1,)), ((), ())),
            preferred_element_type=jnp.float32)

        @pl.when(i == 0)
        def _():
            h_ref[...] = part + b1_ref[...]

        @pl.when(i > 0)
        def _():
            h_ref[...] += part

        @pl.when(i == nb1 - 1)
        def _():
            h_ref[...] = jnp.maximum(h_ref[...], 0.0)

    @pl.when(i >= nb1)
    def _():
        j = i - nb1
        logits = jax.lax.dot_general(
            h_ref[...], w2_ref[...], (((1,), (1,)), ((), ())),
            preferred_element_type=jnp.float32) + b2_ref[0]
        out_ref[pl.ds(j, 1)] = logits[None]
        bm = jnp.max(logits)
        bs = jnp.sum(jnp.exp(logits - bm))

        @pl.when(j == 0)
        def _():
            acc_ref[0] = bm
            acc_ref[1] = bs

        @pl.when(j > 0)
        def _():
            m_old = acc_ref[0]
            s_old = acc_ref[1]
            m_new = jnp.maximum(m_old, bm)
            acc_ref[0] = m_new
            acc_ref[1] = s_old * jnp.exp(m_old - m_new) + bs * jnp.exp(bm - m_new)

        @pl.when(j == nb2 - 1)
        def _():
            lse = jnp.log(acc_ref[1]) + acc_ref[0]
            out_ref[...] = out_ref[...] - lse


def _mlp(emb_flat, W1, b1, W2, b2, nb1=8, nb2=20):
    hid, k = W1.shape
    vocab = W2.shape[0]
    bk = k // nb1
    bv = vocab // nb2

    out = pl.pallas_call(
        functools.partial(_mlp_body, nb1, nb2),
        grid=(nb1 + nb2,),
        in_specs=[
            pl.BlockSpec((1, bk), lambda i: (0, jnp.minimum(i, nb1 - 1))),
            pl.BlockSpec((hid, bk), lambda i: (0, jnp.minimum(i, nb1 - 1))),
            pl.BlockSpec((1, hid), lambda i: (0, 0)),
            pl.BlockSpec((bv, hid), lambda i: (jnp.maximum(i - nb1, 0), 0)),
            pl.BlockSpec((1, 1, bv), lambda i: (jnp.maximum(i - nb1, 0), 0, 0)),
        ],
        out_specs=pl.BlockSpec((nb2, 1, bv), lambda i: (0, 0, 0)),
        out_shape=jax.ShapeDtypeStruct((nb2, 1, bv), jnp.float32),
        scratch_shapes=[
            pltpu.VMEM((1, hid), jnp.float32),
            pltpu.SMEM((2,), jnp.float32),
        ],
    )(emb_flat, W1, b1.reshape(1, hid), W2, b2.reshape(nb2, 1, bv))
    return out.reshape(1, vocab)


def kernel(words, table, W1, b1, W2, b2):
    ctx = words.shape[0]
    emb_dim = table.shape[1]
    emb = _sc_gather(table, words.astype(jnp.int32), 256, emb_dim)
    emb_flat = emb.reshape(1, ctx * emb_dim)
    return _mlp(emb_flat, W1, b1, W2, b2, nb1=4, nb2=10)


# speedup vs baseline: 1.0527x; 1.0101x over previous
import functools

import jax
import jax.numpy as jnp
from jax.experimental import pallas as pl
from jax.experimental.pallas import tpu as pltpu
from jax.experimental.pallas import tpu_sc as plsc


def _sc_gather(table, idx, b_pad, emb_dim):
    info = plsc.get_sparse_core_info()
    nc, ns = info.num_cores, info.num_subcores
    nw = nc * ns
    b_per_w = b_pad // nw
    n_rows = idx.shape[0]
    mesh = plsc.VectorSubcoreMesh(core_axis_name="c", subcore_axis_name="s")

    @functools.partial(
        pl.kernel,
        mesh=mesh,
        out_type=jax.ShapeDtypeStruct((n_rows, emb_dim), jnp.float32),
        scratch_types=[
            pltpu.VMEM((b_per_w,), jnp.int32),
            pltpu.VMEM((b_per_w, emb_dim), jnp.float32),
            pltpu.SemaphoreType.DMA,
        ],
    )
    def gather_k(table_hbm, idx_hbm, out_hbm, idx_v, rows_v, sem):
        wid = jax.lax.axis_index("s") * nc + jax.lax.axis_index("c")
        base = wid * b_per_w

        @pl.when(base < n_rows)
        def _():
            pltpu.sync_copy(idx_hbm.at[pl.ds(base, b_per_w)], idx_v)
            pltpu.async_copy(table_hbm.at[idx_v], rows_v, sem).wait()
            pltpu.sync_copy(rows_v, out_hbm.at[pl.ds(base, b_per_w)])

    return gather_k(table, idx)


def _mlp_body(nb1, nb2, x_ref, w1_ref, b1_ref, w2_ref, b2_ref, out_ref,
              h_ref, acc_ref):
    i = pl.program_id(0)

    @pl.when(i < nb1)
    def _():
        part = jax.lax.dot_general(
            x_ref[...], w1_ref[...], (((1,), (1,)), ((), ())),
            preferred_element_type=jnp.float32)

        @pl.when(i == 0)
        def _():
            h_ref[...] = part + b1_ref[...]

        @pl.when(i > 0)
        def _():
            h_ref[...] += part

        @pl.when(i == nb1 - 1)
        def _():
            h_ref[...] = jnp.maximum(h_ref[...], 0.0)

    @pl.when(i >= nb1)
    def _():
        j = i - nb1
        logits = jax.lax.dot_general(
            h_ref[...], w2_ref[...], (((1,), (1,)), ((), ())),
            preferred_element_type=jnp.float32) + b2_ref[0]
        out_ref[pl.ds(j, 1)] = logits[None]
        bm = jnp.max(logits)
        bs = jnp.sum(jnp.exp(logits - bm))

        @pl.when(j == 0)
        def _():
            acc_ref[0] = bm
            acc_ref[1] = bs

        @pl.when(j > 0)
        def _():
            m_old = acc_ref[0]
            s_old = acc_ref[1]
            m_new = jnp.maximum(m_old, bm)
            acc_ref[0] = m_new
            acc_ref[1] = s_old * jnp.exp(m_old - m_new) + bs * jnp.exp(bm - m_new)

        @pl.when(j == nb2 - 1)
        def _():
            lse = jnp.log(acc_ref[1]) + acc_ref[0]
            out_ref[...] = out_ref[...] - lse


def _mlp(emb_flat, W1, b1, W2, b2, nb1=8, nb2=20):
    hid, k = W1.shape
    vocab = W2.shape[0]
    bk = k // nb1
    bv = vocab // nb2

    out = pl.pallas_call(
        functools.partial(_mlp_body, nb1, nb2),
        grid=(nb1 + nb2,),
        in_specs=[
            pl.BlockSpec((1, bk), lambda i: (0, jnp.minimum(i, nb1 - 1))),
            pl.BlockSpec((hid, bk), lambda i: (0, jnp.minimum(i, nb1 - 1))),
            pl.BlockSpec((1, hid), lambda i: (0, 0)),
            pl.BlockSpec((bv, hid), lambda i: (jnp.maximum(i - nb1, 0), 0)),
            pl.BlockSpec((1, 1, bv), lambda i: (jnp.maximum(i - nb1, 0), 0, 0)),
        ],
        out_specs=pl.BlockSpec((nb2, 1, bv), lambda i: (0, 0, 0)),
        out_shape=jax.ShapeDtypeStruct((nb2, 1, bv), jnp.float32),
        scratch_shapes=[
            pltpu.VMEM((1, hid), jnp.float32),
            pltpu.SMEM((2,), jnp.float32),
        ],
    )(emb_flat, W1, b1.reshape(1, hid), W2, b2.reshape(nb2, 1, bv))
    return out.reshape(1, vocab)


def kernel(words, table, W1, b1, W2, b2):
    ctx = words.shape[0]
    emb_dim = table.shape[1]
    emb = _sc_gather(table, words.astype(jnp.int32), 256, emb_dim)
    emb_flat = emb.reshape(1, ctx * emb_dim)
    return _mlp(emb_flat, W1, b1, W2, b2, nb1=2, nb2=10)
